# Initial kernel scaffold; baseline (speedup 1.0000x reference)
import jax
import jax.numpy as jnp
from jax.experimental import pallas as pl


def kernel(prev_cell, prev_hidden, child_indices):
    raise NotImplementedError("write your pallas kernel here")


# speedup vs baseline: 5.2481x; 5.2481x over previous
import functools

import jax
import jax.numpy as jnp
from jax import lax
from jax.experimental import pallas as pl
from jax.experimental.pallas import tpu as pltpu
from jax.experimental.pallas import tpu_sc as plsc

NC, NS = 2, 16
NW = NC * NS
B = 320000
D = 128
BPW = B // NW
C = 200
NCHUNK = BPW // C


def _gather_kernel(cell_hbm, hid_hbm, idx_hbm, out_cell, out_hid,
                   idx_v, cell_v, hid_v, sem_c, sem_h):
    wid = lax.axis_index("s") * NC + lax.axis_index("c")
    base = wid * BPW

    @pl.loop(0, NCHUNK)
    def _(g):
        off = pl.multiple_of(base + g * C, 8)
        pltpu.sync_copy(idx_hbm.at[pl.ds(off, C)], idx_v)
        cp_c = pltpu.async_copy(cell_hbm.at[idx_v], cell_v, sem_c)
        cp_h = pltpu.async_copy(hid_hbm.at[idx_v], hid_v, sem_h)
        cp_c.wait()
        pltpu.sync_copy(cell_v, out_cell.at[pl.ds(off, C)])
        cp_h.wait()
        pltpu.sync_copy(hid_v, out_hid.at[pl.ds(off, C)])


def kernel(prev_cell, prev_hidden, child_indices):
    mesh = plsc.VectorSubcoreMesh(core_axis_name="c", subcore_axis_name="s")
    run = functools.partial(
        pl.kernel,
        out_type=(
            jax.ShapeDtypeStruct((B, D), jnp.float32),
            jax.ShapeDtypeStruct((B, D), jnp.float32),
        ),
        mesh=mesh,
        scratch_types=[
            pltpu.VMEM((C,), jnp.int32),
            pltpu.VMEM((C, D), jnp.float32),
            pltpu.VMEM((C, D), jnp.float32),
            pltpu.SemaphoreType.DMA,
            pltpu.SemaphoreType.DMA,
        ],
    )(_gather_kernel)
    return run(prev_cell, prev_hidden, child_indices.astype(jnp.int32))


# speedup vs baseline: 6.3095x; 1.2022x over previous
import functools

import jax
import jax.numpy as jnp
from jax import lax
from jax.experimental import pallas as pl
from jax.experimental.pallas import tpu as pltpu
from jax.experimental.pallas import tpu_sc as plsc

NC, NS = 2, 16
NW = NC * NS
B = 320000
D = 128
BPW = B // NW
C = 200
NCHUNK = BPW // C


def _gather_kernel(cell_hbm, hid_hbm, idx_hbm, out_cell, out_hid,
                   idx0, idx1, cell0, cell1, hid0, hid1,
                   sc0, sc1, sh0, sh1):
    wid = lax.axis_index("s") * NC + lax.axis_index("c")
    base = wid * BPW
    bufs = ((idx0, cell0, hid0, sc0, sh0), (idx1, cell1, hid1, sc1, sh1))

    def fire(chunk, b):
        idx_v, cell_v, hid_v, sem_c, sem_h = bufs[b]
        off = pl.multiple_of(base + chunk * C, 8)
        pltpu.sync_copy(idx_hbm.at[pl.ds(off, C)], idx_v)
        pltpu.async_copy(cell_hbm.at[idx_v], cell_v, sem_c)
        pltpu.async_copy(hid_hbm.at[idx_v], hid_v, sem_h)

    def drain(chunk, b):
        idx_v, cell_v, hid_v, sem_c, sem_h = bufs[b]
        off = pl.multiple_of(base + chunk * C, 8)
        pltpu.make_async_copy(cell_hbm.at[idx_v], cell_v, sem_c).wait()
        pltpu.sync_copy(cell_v, out_cell.at[pl.ds(off, C)])
        pltpu.make_async_copy(hid_hbm.at[idx_v], hid_v, sem_h).wait()
        pltpu.sync_copy(hid_v, out_hid.at[pl.ds(off, C)])

    fire(0, 0)

    @pl.loop(0, NCHUNK - 2, step=2)
    def _(g):
        fire(g + 1, 1)
        drain(g, 0)
        fire(g + 2, 0)
        drain(g + 1, 1)

    fire(NCHUNK - 1, 1)
    drain(NCHUNK - 2, 0)
    drain(NCHUNK - 1, 1)


def kernel(prev_cell, prev_hidden, child_indices):
    mesh = plsc.VectorSubcoreMesh(core_axis_name="c", subcore_axis_name="s")
    run = functools.partial(
        pl.kernel,
        out_type=(
            jax.ShapeDtypeStruct((B, D), jnp.float32),
            jax.ShapeDtypeStruct((B, D), jnp.float32),
        ),
        mesh=mesh,
        scratch_types=[
            pltpu.VMEM((C,), jnp.int32),
            pltpu.VMEM((C,), jnp.int32),
            pltpu.VMEM((C, D), jnp.float32),
            pltpu.VMEM((C, D), jnp.float32),
            pltpu.VMEM((C, D), jnp.float32),
            pltpu.VMEM((C, D), jnp.float32),
            pltpu.SemaphoreType.DMA,
            pltpu.SemaphoreType.DMA,
            pltpu.SemaphoreType.DMA,
            pltpu.SemaphoreType.DMA,
        ],
    )(_gather_kernel)
    return run(prev_cell, prev_hidden, child_indices.astype(jnp.int32))
